# initial kernel scaffold (unmeasured)
import jax
import jax.numpy as jnp
from jax import lax
from jax.experimental import pallas as pl
from jax.experimental.pallas import tpu as pltpu

N_DEV = 8
B, SQ, SKV, D_MODEL = 2, 256, 256, 512
H_PER = 4
DH = 64
WINDOW = 128


def kernel(x, Wq, K_ext, V_ext, Wo):
    my = lax.axis_index("i")

    K = lax.dynamic_slice_in_dim(K_ext, my * H_PER, H_PER, axis=2)
    V = lax.dynamic_slice_in_dim(V_ext, my * H_PER, H_PER, axis=2)
    K = jnp.transpose(K, (0, 2, 1, 3)).astype(jnp.bfloat16)
    V = jnp.transpose(V, (0, 2, 1, 3)).astype(jnp.bfloat16)

    xb = x.astype(jnp.bfloat16)
    Wqb = Wq.astype(jnp.bfloat16)
    Wob = Wo.astype(jnp.bfloat16)

    def body(x_ref, wq_ref, k_ref, v_ref, wo_ref, out_ref,
             comm_ref, send_sems, recv_sems):
        left = (my - 1) % N_DEV
        right = (my + 1) % N_DEV

        barrier_sem = pltpu.get_barrier_semaphore()
        for nbr in [left, right]:
            pl.semaphore_signal(
                barrier_sem, inc=1,
                device_id=(nbr,), device_id_type=pl.DeviceIdType.MESH,
            )
        pl.semaphore_wait(barrier_sem, 2)

        qi = lax.broadcasted_iota(jnp.int32, (SQ, SKV), 0)
        ki = lax.broadcasted_iota(jnp.int32, (SQ, SKV), 1)
        mask = jnp.abs(qi - ki) <= WINDOW

        for b in range(B):
            q = jnp.dot(x_ref[b], wq_ref[...],
                        preferred_element_type=jnp.float32)
            ctx_cols = []
            for h in range(H_PER):
                qh = (q[:, DH * h:DH * (h + 1)] * 0.125).astype(jnp.bfloat16)
                s = lax.dot_general(
                    qh, k_ref[b, h],
                    (((1,), (1,)), ((), ())),
                    preferred_element_type=jnp.float32)
                s = jnp.where(mask, s, -1e9)
                m = jnp.max(s, axis=-1, keepdims=True)
                w = jnp.exp(s - m)
                w = w / jnp.sum(w, axis=-1, keepdims=True)
                ctx_cols.append(
                    jnp.dot(w.astype(jnp.bfloat16), v_ref[b, h],
                            preferred_element_type=jnp.float32))
            ctx = jnp.concatenate(ctx_cols, axis=1)
            partial = jnp.dot(ctx.astype(jnp.bfloat16), wo_ref[...],
                              preferred_element_type=jnp.float32)
            out_ref[b, :, :] = partial
            comm_ref[0, SQ * b:SQ * (b + 1), :] = partial.astype(jnp.bfloat16)

        for h in range(N_DEV - 1):
            rdma = pltpu.make_async_remote_copy(
                src_ref=comm_ref.at[h],
                dst_ref=comm_ref.at[h + 1],
                send_sem=send_sems.at[h],
                recv_sem=recv_sems.at[h],
                device_id=(right,),
                device_id_type=pl.DeviceIdType.MESH,
            )
            rdma.start()
            rdma.wait()
            for b in range(B):
                out_ref[b, :, :] += comm_ref[
                    h + 1, SQ * b:SQ * (b + 1), :].astype(jnp.float32)

    return pl.pallas_call(
        body,
        out_shape=jax.ShapeDtypeStruct((B, SQ, D_MODEL), jnp.float32),
        in_specs=[pl.BlockSpec(memory_space=pltpu.VMEM)] * 5,
        out_specs=pl.BlockSpec(memory_space=pltpu.VMEM),
        scratch_shapes=[
            pltpu.VMEM((N_DEV, B * SQ, D_MODEL), jnp.bfloat16),
            pltpu.SemaphoreType.DMA((N_DEV - 1,)),
            pltpu.SemaphoreType.DMA((N_DEV - 1,)),
        ],
        compiler_params=pltpu.CompilerParams(collective_id=0),
    )(xb, Wqb, K, V, Wob)


# baseline (device time: 63909 ns/iter reference)
import jax
import jax.numpy as jnp
from jax import lax
from jax.experimental import pallas as pl
from jax.experimental.pallas import tpu as pltpu

N_DEV = 8
B, SQ, SKV, D_MODEL = 2, 256, 256, 512
H_PER = 4
DH = 64
WINDOW = 128


def kernel(x, Wq, K_ext, V_ext, Wo):
    my = lax.axis_index("i")

    K = lax.dynamic_slice_in_dim(K_ext, my * H_PER, H_PER, axis=2)
    V = lax.dynamic_slice_in_dim(V_ext, my * H_PER, H_PER, axis=2)
    K = jnp.transpose(K, (0, 2, 1, 3)).astype(jnp.bfloat16)
    V = jnp.transpose(V, (0, 2, 1, 3)).astype(jnp.bfloat16)

    xb = x.astype(jnp.bfloat16)
    Wqb = Wq.astype(jnp.bfloat16)
    Wob = Wo.astype(jnp.bfloat16)

    def body(x_ref, wq_ref, k_ref, v_ref, wo_ref, out_ref,
             comm_ref, send_sems, recv_sems):
        pos = lax.axis_index("i")
        left = (pos - 1) % N_DEV
        right = (pos + 1) % N_DEV

        barrier_sem = pltpu.get_barrier_semaphore()
        for nbr in [left, right]:
            pl.semaphore_signal(
                barrier_sem, inc=1,
                device_id=(nbr,), device_id_type=pl.DeviceIdType.MESH,
            )
        pl.semaphore_wait(barrier_sem, 2)

        qi = lax.broadcasted_iota(jnp.int32, (SQ, SKV), 0)
        ki = lax.broadcasted_iota(jnp.int32, (SQ, SKV), 1)
        mask = jnp.abs(qi - ki) <= WINDOW

        for b in range(B):
            q = jnp.dot(x_ref[b], wq_ref[...],
                        preferred_element_type=jnp.float32)
            ctx_cols = []
            for h in range(H_PER):
                qh = (q[:, DH * h:DH * (h + 1)] * 0.125).astype(jnp.bfloat16)
                s = lax.dot_general(
                    qh, k_ref[b, h],
                    (((1,), (1,)), ((), ())),
                    preferred_element_type=jnp.float32)
                s = jnp.where(mask, s, -1e9)
                m = jnp.max(s, axis=-1, keepdims=True)
                w = jnp.exp(s - m)
                w = w / jnp.sum(w, axis=-1, keepdims=True)
                ctx_cols.append(
                    jnp.dot(w.astype(jnp.bfloat16), v_ref[b, h],
                            preferred_element_type=jnp.float32))
            ctx = jnp.concatenate(ctx_cols, axis=1)
            partial = jnp.dot(ctx.astype(jnp.bfloat16), wo_ref[...],
                              preferred_element_type=jnp.float32)
            out_ref[b, :, :] = partial
            comm_ref[0, SQ * b:SQ * (b + 1), :] = partial.astype(jnp.bfloat16)

        for h in range(N_DEV - 1):
            rdma = pltpu.make_async_remote_copy(
                src_ref=comm_ref.at[h],
                dst_ref=comm_ref.at[h + 1],
                send_sem=send_sems.at[h],
                recv_sem=recv_sems.at[h],
                device_id=(right,),
                device_id_type=pl.DeviceIdType.MESH,
            )
            rdma.start()
            rdma.wait()
            for b in range(B):
                out_ref[b, :, :] += comm_ref[
                    h + 1, SQ * b:SQ * (b + 1), :].astype(jnp.float32)

    return pl.pallas_call(
        body,
        out_shape=jax.ShapeDtypeStruct((B, SQ, D_MODEL), jnp.float32),
        in_specs=[pl.BlockSpec(memory_space=pltpu.VMEM)] * 5,
        out_specs=pl.BlockSpec(memory_space=pltpu.VMEM),
        scratch_shapes=[
            pltpu.VMEM((N_DEV, B * SQ, D_MODEL), jnp.bfloat16),
            pltpu.SemaphoreType.DMA((N_DEV - 1,)),
            pltpu.SemaphoreType.DMA((N_DEV - 1,)),
        ],
        compiler_params=pltpu.CompilerParams(collective_id=0),
    )(xb, Wqb, K, V, Wob)


# device time: 34596 ns/iter; 1.8473x vs baseline; 1.8473x over previous
import jax
import jax.numpy as jnp
from jax import lax
from jax.experimental import pallas as pl
from jax.experimental.pallas import tpu as pltpu

N_DEV = 8
B, SQ, SKV, D_MODEL = 2, 256, 256, 512
H_PER = 4
DH = 64
WINDOW = 128


def kernel(x, Wq, K_ext, V_ext, Wo):
    my = lax.axis_index("i")

    K = lax.dynamic_slice_in_dim(K_ext, my * H_PER, H_PER, axis=2)
    V = lax.dynamic_slice_in_dim(V_ext, my * H_PER, H_PER, axis=2)
    K = jnp.transpose(K, (0, 2, 1, 3)).astype(jnp.bfloat16)
    V = jnp.transpose(V, (0, 2, 1, 3)).astype(jnp.bfloat16)

    xb = x.astype(jnp.bfloat16)
    Wqb = Wq.astype(jnp.bfloat16)
    Wob = Wo.astype(jnp.bfloat16)

    def body(x_ref, wq_ref, k_ref, v_ref, wo_ref, out_ref,
             comm_ref, send_sems, recv_sems):
        pos = lax.axis_index("i")
        partners = [pos ^ (1 << k) for k in range(3)]

        barrier_sem = pltpu.get_barrier_semaphore()
        for nbr in partners:
            pl.semaphore_signal(
                barrier_sem, inc=1,
                device_id=(nbr,), device_id_type=pl.DeviceIdType.MESH,
            )
        pl.semaphore_wait(barrier_sem, 3)

        qi = lax.broadcasted_iota(jnp.int32, (SQ, SKV), 0)
        ki = lax.broadcasted_iota(jnp.int32, (SQ, SKV), 1)
        mask = jnp.abs(qi - ki) <= WINDOW

        for b in range(B):
            q = jnp.dot(x_ref[b], wq_ref[...],
                        preferred_element_type=jnp.float32)
            ctx_cols = []
            for h in range(H_PER):
                qh = (q[:, DH * h:DH * (h + 1)] * 0.125).astype(jnp.bfloat16)
                s = lax.dot_general(
                    qh, k_ref[b, h],
                    (((1,), (1,)), ((), ())),
                    preferred_element_type=jnp.float32)
                s = jnp.where(mask, s, -1e9)
                m = jnp.max(s, axis=-1, keepdims=True)
                w = jnp.exp(s - m)
                w = w / jnp.sum(w, axis=-1, keepdims=True)
                ctx_cols.append(
                    jnp.dot(w.astype(jnp.bfloat16), v_ref[b, h],
                            preferred_element_type=jnp.float32))
            ctx = jnp.concatenate(ctx_cols, axis=1)
            partial = jnp.dot(ctx.astype(jnp.bfloat16), wo_ref[...],
                              preferred_element_type=jnp.float32)
            out_ref[b, :, :] = partial
            comm_ref[0, SQ * b:SQ * (b + 1), :] = partial.astype(jnp.bfloat16)

        for k in range(3):
            rdma = pltpu.make_async_remote_copy(
                src_ref=comm_ref.at[2 * k],
                dst_ref=comm_ref.at[2 * k + 1],
                send_sem=send_sems.at[k],
                recv_sem=recv_sems.at[k],
                device_id=(partners[k],),
                device_id_type=pl.DeviceIdType.MESH,
            )
            rdma.start()
            rdma.wait()
            if k < 2:
                comm_ref[2 * k + 2, :, :] = (
                    comm_ref[2 * k, :, :] + comm_ref[2 * k + 1, :, :])
            for b in range(B):
                out_ref[b, :, :] += comm_ref[
                    2 * k + 1, SQ * b:SQ * (b + 1), :].astype(jnp.float32)

    return pl.pallas_call(
        body,
        out_shape=jax.ShapeDtypeStruct((B, SQ, D_MODEL), jnp.float32),
        in_specs=[pl.BlockSpec(memory_space=pltpu.VMEM)] * 5,
        out_specs=pl.BlockSpec(memory_space=pltpu.VMEM),
        scratch_shapes=[
            pltpu.VMEM((6, B * SQ, D_MODEL), jnp.bfloat16),
            pltpu.SemaphoreType.DMA((3,)),
            pltpu.SemaphoreType.DMA((3,)),
        ],
        compiler_params=pltpu.CompilerParams(collective_id=0),
    )(xb, Wqb, K, V, Wob)


# device time: 23082 ns/iter; 2.7688x vs baseline; 1.4988x over previous
import jax
import jax.numpy as jnp
from jax import lax
from jax.experimental import pallas as pl
from jax.experimental.pallas import tpu as pltpu

N_DEV = 8
B, SQ, SKV, D_MODEL = 2, 256, 256, 512
H_PER = 4
DH = 64
WINDOW = 128
ROWS = B * SQ
CHUNK = ROWS // N_DEV


def kernel(x, Wq, K_ext, V_ext, Wo):
    my = lax.axis_index("i")

    K = lax.dynamic_slice_in_dim(K_ext, my * H_PER, H_PER, axis=2)
    V = lax.dynamic_slice_in_dim(V_ext, my * H_PER, H_PER, axis=2)
    K = jnp.transpose(K, (0, 2, 1, 3)).astype(jnp.bfloat16)
    V = jnp.transpose(V, (0, 2, 1, 3)).astype(jnp.bfloat16)

    xb = x.astype(jnp.bfloat16)
    Wqb = Wq.astype(jnp.bfloat16)
    Wob = Wo.astype(jnp.bfloat16)

    def body(x_ref, wq_ref, k_ref, v_ref, wo_ref, out_ref,
             pbuf, rs_buf, ag_buf, red_ref,
             ss1, rs1, ss2, rs2):
        pos = lax.axis_index("i")

        barrier_sem = pltpu.get_barrier_semaphore()
        for q in range(1, N_DEV):
            pl.semaphore_signal(
                barrier_sem, inc=1,
                device_id=((pos + q) % N_DEV,),
                device_id_type=pl.DeviceIdType.MESH,
            )
        pl.semaphore_wait(barrier_sem, N_DEV - 1)

        qi = lax.broadcasted_iota(jnp.int32, (SQ, SKV), 0)
        ki = lax.broadcasted_iota(jnp.int32, (SQ, SKV), 1)
        mask = jnp.abs(qi - ki) <= WINDOW

        for b in range(B):
            q = jnp.dot(x_ref[b], wq_ref[...],
                        preferred_element_type=jnp.float32)
            ctx_cols = []
            for h in range(H_PER):
                qh = (q[:, DH * h:DH * (h + 1)] * 0.125).astype(jnp.bfloat16)
                s = lax.dot_general(
                    qh, k_ref[b, h],
                    (((1,), (1,)), ((), ())),
                    preferred_element_type=jnp.float32)
                s = jnp.where(mask, s, -1e9)
                m = jnp.max(s, axis=-1, keepdims=True)
                w = jnp.exp(s - m)
                w = w / jnp.sum(w, axis=-1, keepdims=True)
                ctx_cols.append(
                    jnp.dot(w.astype(jnp.bfloat16), v_ref[b, h],
                            preferred_element_type=jnp.float32))
            ctx = jnp.concatenate(ctx_cols, axis=1)
            partial = jnp.dot(ctx.astype(jnp.bfloat16), wo_ref[...],
                              preferred_element_type=jnp.float32)
            pbuf[SQ * b:SQ * (b + 1), :] = partial.astype(jnp.bfloat16)

        p1 = []
        for r in range(1, N_DEV):
            t = (pos + r) % N_DEV
            rdma = pltpu.make_async_remote_copy(
                src_ref=pbuf.at[pl.ds(t * CHUNK, CHUNK), :],
                dst_ref=rs_buf.at[N_DEV - r],
                send_sem=ss1.at[r],
                recv_sem=rs1.at[N_DEV - r],
                device_id=(t,),
                device_id_type=pl.DeviceIdType.MESH,
            )
            rdma.start()
            p1.append(rdma)

        red_ref[...] = pbuf[pl.ds(pos * CHUNK, CHUNK), :].astype(jnp.float32)
        for q in range(1, N_DEV):
            recv = pltpu.make_async_remote_copy(
                src_ref=pbuf.at[pl.ds(0, CHUNK), :],
                dst_ref=rs_buf.at[q],
                send_sem=ss1.at[q],
                recv_sem=rs1.at[q],
                device_id=((pos + q) % N_DEV,),
                device_id_type=pl.DeviceIdType.MESH,
            )
            recv.wait_recv()
            red_ref[...] += rs_buf[q].astype(jnp.float32)
        for rdma in p1:
            rdma.wait_send()

        redb = red_ref[...].astype(jnp.bfloat16)
        ag_buf[0, :, :] = redb

        p2 = []
        for r in range(1, N_DEV):
            t = (pos + r) % N_DEV
            rdma = pltpu.make_async_remote_copy(
                src_ref=ag_buf.at[0],
                dst_ref=ag_buf.at[N_DEV - r],
                send_sem=ss2.at[r],
                recv_sem=rs2.at[N_DEV - r],
                device_id=(t,),
                device_id_type=pl.DeviceIdType.MESH,
            )
            rdma.start()
            p2.append(rdma)

        out_ref[pl.ds(pos * CHUNK, CHUNK), :] = red_ref[...]
        for q in range(1, N_DEV):
            recv = pltpu.make_async_remote_copy(
                src_ref=ag_buf.at[0],
                dst_ref=ag_buf.at[q],
                send_sem=ss2.at[q],
                recv_sem=rs2.at[q],
                device_id=((pos + q) % N_DEV,),
                device_id_type=pl.DeviceIdType.MESH,
            )
            recv.wait_recv()
            s = (pos + q) % N_DEV
            out_ref[pl.ds(s * CHUNK, CHUNK), :] = ag_buf[q].astype(jnp.float32)
        for rdma in p2:
            rdma.wait_send()

    out_flat = pl.pallas_call(
        body,
        out_shape=jax.ShapeDtypeStruct((ROWS, D_MODEL), jnp.float32),
        in_specs=[pl.BlockSpec(memory_space=pltpu.VMEM)] * 5,
        out_specs=pl.BlockSpec(memory_space=pltpu.VMEM),
        scratch_shapes=[
            pltpu.VMEM((ROWS, D_MODEL), jnp.bfloat16),
            pltpu.VMEM((N_DEV, CHUNK, D_MODEL), jnp.bfloat16),
            pltpu.VMEM((N_DEV, CHUNK, D_MODEL), jnp.bfloat16),
            pltpu.VMEM((CHUNK, D_MODEL), jnp.float32),
            pltpu.SemaphoreType.DMA((N_DEV,)),
            pltpu.SemaphoreType.DMA((N_DEV,)),
            pltpu.SemaphoreType.DMA((N_DEV,)),
            pltpu.SemaphoreType.DMA((N_DEV,)),
        ],
        compiler_params=pltpu.CompilerParams(collective_id=0),
    )(xb, Wqb, K, V, Wob)
    return out_flat.reshape(B, SQ, D_MODEL)


# device time: 19529 ns/iter; 3.2725x vs baseline; 1.1819x over previous
import jax
import jax.numpy as jnp
from jax import lax
from jax.experimental import pallas as pl
from jax.experimental.pallas import tpu as pltpu

N_DEV = 8
B, SQ, SKV, D_MODEL = 2, 256, 256, 512
H_PER = 4
DH = 64
WINDOW = 128
ROWS = B * SQ
CHUNK = ROWS // N_DEV
CPB = SQ // CHUNK


def kernel(x, Wq, K_ext, V_ext, Wo):
    my = lax.axis_index("i")

    K = lax.dynamic_slice_in_dim(K_ext, my * H_PER, H_PER, axis=2)
    V = lax.dynamic_slice_in_dim(V_ext, my * H_PER, H_PER, axis=2)
    K = jnp.transpose(K, (0, 2, 1, 3)).astype(jnp.bfloat16)
    V = jnp.transpose(V, (0, 2, 1, 3)).astype(jnp.bfloat16)

    xb = x.astype(jnp.bfloat16)
    Wqb = Wq.astype(jnp.bfloat16)
    Wob = Wo.astype(jnp.bfloat16)

    def body(x_ref, wq_ref, k_ref, v_ref, wo_ref, out_ref,
             pbuf, rs_buf, ag_buf, red_ref, redb_ref,
             ss1, rs1, ss2, rs2):
        pos = lax.axis_index("i")

        barrier_sem = pltpu.get_barrier_semaphore()
        for q in range(1, N_DEV):
            pl.semaphore_signal(
                barrier_sem, inc=1,
                device_id=((pos + q) % N_DEV,),
                device_id_type=pl.DeviceIdType.MESH,
            )
        pl.semaphore_wait(barrier_sem, N_DEV - 1)

        qi = lax.broadcasted_iota(jnp.int32, (SQ, SKV), 0)
        ki = lax.broadcasted_iota(jnp.int32, (SQ, SKV), 1)
        mask = jnp.abs(qi - ki) <= WINDOW

        def send_chunk(c):
            return pltpu.make_async_remote_copy(
                src_ref=pbuf.at[pl.ds(c * CHUNK, CHUNK), :],
                dst_ref=rs_buf.at[pos],
                send_sem=ss1.at[c],
                recv_sem=rs1.at[pos],
                device_id=(c,),
                device_id_type=pl.DeviceIdType.MESH,
            )

        for b in range(B):
            q = jnp.dot(x_ref[b], wq_ref[...],
                        preferred_element_type=jnp.float32)
            ctx_cols = []
            for h in range(H_PER):
                qh = (q[:, DH * h:DH * (h + 1)] * 0.125).astype(jnp.bfloat16)
                s = lax.dot_general(
                    qh, k_ref[b, h],
                    (((1,), (1,)), ((), ())),
                    preferred_element_type=jnp.float32)
                s = jnp.where(mask, s, -1e9)
                m = jnp.max(s, axis=-1, keepdims=True)
                w = jnp.exp(s - m)
                w = w / jnp.sum(w, axis=-1, keepdims=True)
                ctx_cols.append(
                    jnp.dot(w.astype(jnp.bfloat16), v_ref[b, h],
                            preferred_element_type=jnp.float32))
            ctx = jnp.concatenate(ctx_cols, axis=1)
            partial = jnp.dot(ctx.astype(jnp.bfloat16), wo_ref[...],
                              preferred_element_type=jnp.float32)
            pbuf[SQ * b:SQ * (b + 1), :] = partial.astype(jnp.bfloat16)
            for c in range(CPB * b, CPB * (b + 1)):
                @pl.when(c != pos)
                def _(c=c):
                    send_chunk(c).start()

        red_ref[...] = pbuf[pl.ds(pos * CHUNK, CHUNK), :].astype(jnp.float32)
        for s in range(N_DEV):
            @pl.when(s != pos)
            def _(s=s):
                recv = pltpu.make_async_remote_copy(
                    src_ref=pbuf.at[pl.ds(0, CHUNK), :],
                    dst_ref=rs_buf.at[s],
                    send_sem=ss1.at[s],
                    recv_sem=rs1.at[s],
                    device_id=(s,),
                    device_id_type=pl.DeviceIdType.MESH,
                )
                recv.wait_recv()
                red_ref[...] += rs_buf[s].astype(jnp.float32)
        redb_ref[...] = red_ref[...].astype(jnp.bfloat16)

        for t in range(N_DEV):
            @pl.when(t != pos)
            def _(t=t):
                pltpu.make_async_remote_copy(
                    src_ref=redb_ref,
                    dst_ref=ag_buf.at[pos],
                    send_sem=ss2.at[t],
                    recv_sem=rs2.at[pos],
                    device_id=(t,),
                    device_id_type=pl.DeviceIdType.MESH,
                ).start()

        for c in range(N_DEV):
            @pl.when(c != pos)
            def _(c=c):
                send_chunk(c).wait_send()

        out_ref[pl.ds(pos * CHUNK, CHUNK), :] = red_ref[...]

        for s in range(N_DEV):
            @pl.when(s != pos)
            def _(s=s):
                recv = pltpu.make_async_remote_copy(
                    src_ref=redb_ref,
                    dst_ref=ag_buf.at[s],
                    send_sem=ss2.at[s],
                    recv_sem=rs2.at[s],
                    device_id=(s,),
                    device_id_type=pl.DeviceIdType.MESH,
                )
                recv.wait_recv()
                out_ref[s * CHUNK:(s + 1) * CHUNK, :] = (
                    ag_buf[s].astype(jnp.float32))
        for t in range(N_DEV):
            @pl.when(t != pos)
            def _(t=t):
                pltpu.make_async_remote_copy(
                    src_ref=redb_ref,
                    dst_ref=ag_buf.at[pos],
                    send_sem=ss2.at[t],
                    recv_sem=rs2.at[pos],
                    device_id=(t,),
                    device_id_type=pl.DeviceIdType.MESH,
                ).wait_send()

    out_flat = pl.pallas_call(
        body,
        out_shape=jax.ShapeDtypeStruct((ROWS, D_MODEL), jnp.float32),
        in_specs=[pl.BlockSpec(memory_space=pltpu.VMEM)] * 5,
        out_specs=pl.BlockSpec(memory_space=pltpu.VMEM),
        scratch_shapes=[
            pltpu.VMEM((ROWS, D_MODEL), jnp.bfloat16),
            pltpu.VMEM((N_DEV, CHUNK, D_MODEL), jnp.bfloat16),
            pltpu.VMEM((N_DEV, CHUNK, D_MODEL), jnp.bfloat16),
            pltpu.VMEM((CHUNK, D_MODEL), jnp.float32),
            pltpu.VMEM((CHUNK, D_MODEL), jnp.bfloat16),
            pltpu.SemaphoreType.DMA((N_DEV,)),
            pltpu.SemaphoreType.DMA((N_DEV,)),
            pltpu.SemaphoreType.DMA((N_DEV,)),
            pltpu.SemaphoreType.DMA((N_DEV,)),
        ],
        compiler_params=pltpu.CompilerParams(collective_id=0),
    )(xb, Wqb, K, V, Wob)
    return out_flat.reshape(B, SQ, D_MODEL)
